# baseline (speedup 1.0000x reference)
import functools

import jax
import jax.numpy as jnp
from jax import lax
from jax.experimental import pallas as pl
from jax.experimental.pallas import tpu as pltpu
from jax.experimental.pallas import tpu_sc as plsc

N = 8192
SC_ROWS = 0


TC_BR = 128
TC_BC = 8192


def _tc_body(emb_ref, r_ref, val_ref, arg_ref):
    r = pl.program_id(1)

    @pl.when(r == 0)
    def _init():
        val_ref[...] = jnp.zeros_like(val_ref)
        arg_ref[...] = jnp.zeros_like(arg_ref)

    emb = emb_ref[0, :]
    blk = r_ref[...]
    p = blk * emb[:, None]
    m = jnp.max(p, axis=0)
    rows = lax.broadcasted_iota(jnp.int32, p.shape, 0)
    cand = jnp.where(p == m[None, :], rows, N)
    a = jnp.min(cand, axis=0) + (r * TC_BR + SC_ROWS)
    cur = val_ref[0, :]
    upd = m > cur
    val_ref[0, :] = jnp.where(upd, m, cur)
    arg_ref[0, :] = jnp.where(upd, a.astype(jnp.float32), arg_ref[0, :])


def _tc_part(embedding, r_embedding):
    rb0 = SC_ROWS // TC_BR
    grid = (N // TC_BC, (N - SC_ROWS) // TC_BR)
    return pl.pallas_call(
        _tc_body,
        grid=grid,
        in_specs=[
            pl.BlockSpec((1, TC_BR), lambda c, r: (0, r + rb0)),
            pl.BlockSpec((TC_BR, TC_BC), lambda c, r: (r + rb0, c)),
        ],
        out_specs=[
            pl.BlockSpec((1, TC_BC), lambda c, r: (0, c)),
            pl.BlockSpec((1, TC_BC), lambda c, r: (0, c)),
        ],
        out_shape=[
            jax.ShapeDtypeStruct((1, N), jnp.float32),
            jax.ShapeDtypeStruct((1, N), jnp.float32),
        ],
    )(embedding, r_embedding)


def _merge_body(sv_ref, sa_ref, tv_ref, ta_ref, val_ref, arg_ref):
    sv, tv = sv_ref[...], tv_ref[...]
    take_sc = sv >= tv
    val_ref[...] = jnp.where(take_sc, sv, tv)
    arg_ref[...] = jnp.where(take_sc, sa_ref[...], ta_ref[...])


def _merge(sv, sa, tv, ta):
    return pl.pallas_call(
        _merge_body,
        out_shape=[
            jax.ShapeDtypeStruct((1, N), jnp.float32),
            jax.ShapeDtypeStruct((1, N), jnp.float32),
        ],
    )(sv, sa, tv, ta)


SC_RB = 128
NW = 32
CW = N // NW


def _sc_compute(buf, emb_v, h0, m, a):
    nj = len(m)

    def g_body(g, carry):
        m, a = carry
        hb = h0 + g * 16
        ev = emb_v[pl.ds(hb, 16)]
        hbf = hb.astype(jnp.float32)
        for k in range(16):
            e = ev[k]
            hf = jnp.full((16,), hbf + float(k), jnp.float32)
            mm, aa = [], []
            for j in range(nj):
                rv = buf[g * 16 + k, pl.ds(j * 16, 16)]
                p = rv * e
                upd = p > m[j]
                mm.append(jnp.where(upd, p, m[j]))
                aa.append(jnp.where(upd, hf, a[j]))
            m, a = tuple(mm), tuple(aa)
        return m, a

    return lax.fori_loop(0, SC_RB // 16, g_body, (m, a))


def _sc_part(emb1d, r_embedding):
    nch = SC_ROWS // SC_RB
    mesh = plsc.VectorSubcoreMesh(core_axis_name="c", subcore_axis_name="s")

    @functools.partial(
        pl.kernel,
        mesh=mesh,
        out_type=[
            jax.ShapeDtypeStruct((N,), jnp.float32),
            jax.ShapeDtypeStruct((N,), jnp.float32),
        ],
        scratch_types=[
            pltpu.VMEM((SC_ROWS,), jnp.float32),
            pltpu.VMEM((2, SC_RB, CW), jnp.float32),
            pltpu.VMEM((CW,), jnp.float32),
            pltpu.VMEM((CW,), jnp.float32),
            pltpu.SemaphoreType.DMA,
            pltpu.SemaphoreType.DMA,
        ],
    )
    def sc_kernel(emb_hbm, r_hbm, val_hbm, arg_hbm,
                  emb_v, rbuf, val_v, arg_v, sem0, sem1):
        wid = lax.axis_index("s") * 2 + lax.axis_index("c")
        c0 = wid * CW
        sems = (sem0, sem1)
        pltpu.sync_copy(emb_hbm.at[pl.ds(0, SC_ROWS)], emb_v)
        for b in range(2):
            pltpu.async_copy(
                r_hbm.at[pl.ds(b * SC_RB, SC_RB), pl.ds(c0, CW)],
                rbuf.at[b], sems[b])

        nj = CW // 16
        m0 = tuple(jnp.zeros((16,), jnp.float32) for _ in range(nj))
        a0 = tuple(jnp.zeros((16,), jnp.float32) for _ in range(nj))

        def pair_body(gp, carry):
            m, a = carry
            for b in range(2):
                g = gp * 2 + b
                pltpu.make_async_copy(
                    r_hbm.at[pl.ds(0, SC_RB), pl.ds(c0, CW)],
                    rbuf.at[b], sems[b]).wait()
                m, a = _sc_compute(rbuf.at[b], emb_v, g * SC_RB, m, a)

                @pl.when(g + 2 < nch)
                def _():
                    pltpu.async_copy(
                        r_hbm.at[pl.ds((g + 2) * SC_RB, SC_RB), pl.ds(c0, CW)],
                        rbuf.at[b], sems[b])
            return m, a

        m, a = lax.fori_loop(0, nch // 2, pair_body, (m0, a0))
        for j in range(nj):
            val_v[pl.ds(j * 16, 16)] = m[j]
            arg_v[pl.ds(j * 16, 16)] = a[j]
        pltpu.sync_copy(val_v, val_hbm.at[pl.ds(c0, CW)])
        pltpu.sync_copy(arg_v, arg_hbm.at[pl.ds(c0, CW)])

    return sc_kernel(emb1d, r_embedding)


def kernel(embedding, r_embedding):
    if SC_ROWS == 0:
        val, arg = _tc_part(embedding, r_embedding)
    elif SC_ROWS == N:
        sval, sarg = _sc_part(embedding.reshape(N), r_embedding)
        val, arg = sval[None, :], sarg[None, :]
    else:
        sval, sarg = _sc_part(embedding.reshape(N), r_embedding)
        tval, targ = _tc_part(embedding, r_embedding)
        val, arg = _merge(sval[None, :], sarg[None, :], tval, targ)
    return val, arg[0]


# speedup vs baseline: 1.2973x; 1.2973x over previous
import functools

import jax
import jax.numpy as jnp
from jax import lax
from jax.experimental import pallas as pl
from jax.experimental.pallas import tpu as pltpu
from jax.experimental.pallas import tpu_sc as plsc

N = 8192
SC_ROWS = 0


TC_BR = 512
TC_BC = 8192


def _tc_body(emb_ref, r_ref, val_ref, arg_ref):
    r = pl.program_id(1)

    @pl.when(r == 0)
    def _init():
        val_ref[...] = jnp.zeros_like(val_ref)
        arg_ref[...] = jnp.zeros_like(arg_ref)

    emb = emb_ref[0, :]
    blk = r_ref[...]
    p = blk * emb[:, None]
    m = jnp.max(p, axis=0)
    rows = lax.broadcasted_iota(jnp.int32, p.shape, 0)
    cand = jnp.where(p == m[None, :], rows, N)
    a = jnp.min(cand, axis=0) + (r * TC_BR + SC_ROWS)
    cur = val_ref[0, :]
    upd = m > cur
    val_ref[0, :] = jnp.where(upd, m, cur)
    arg_ref[0, :] = jnp.where(upd, a.astype(jnp.float32), arg_ref[0, :])


def _tc_part(embedding, r_embedding):
    rb0 = SC_ROWS // TC_BR
    grid = (N // TC_BC, (N - SC_ROWS) // TC_BR)
    return pl.pallas_call(
        _tc_body,
        grid=grid,
        in_specs=[
            pl.BlockSpec((1, TC_BR), lambda c, r: (0, r + rb0)),
            pl.BlockSpec((TC_BR, TC_BC), lambda c, r: (r + rb0, c)),
        ],
        out_specs=[
            pl.BlockSpec((1, TC_BC), lambda c, r: (0, c)),
            pl.BlockSpec((1, TC_BC), lambda c, r: (0, c)),
        ],
        out_shape=[
            jax.ShapeDtypeStruct((1, N), jnp.float32),
            jax.ShapeDtypeStruct((1, N), jnp.float32),
        ],
    )(embedding, r_embedding)


def _merge_body(sv_ref, sa_ref, tv_ref, ta_ref, val_ref, arg_ref):
    sv, tv = sv_ref[...], tv_ref[...]
    take_sc = sv >= tv
    val_ref[...] = jnp.where(take_sc, sv, tv)
    arg_ref[...] = jnp.where(take_sc, sa_ref[...], ta_ref[...])


def _merge(sv, sa, tv, ta):
    return pl.pallas_call(
        _merge_body,
        out_shape=[
            jax.ShapeDtypeStruct((1, N), jnp.float32),
            jax.ShapeDtypeStruct((1, N), jnp.float32),
        ],
    )(sv, sa, tv, ta)


SC_RB = 128
NW = 32
CW = N // NW


def _sc_compute(buf, emb_v, h0, m, a):
    nj = len(m)

    def g_body(g, carry):
        m, a = carry
        hb = h0 + g * 16
        ev = emb_v[pl.ds(hb, 16)]
        hbf = hb.astype(jnp.float32)
        for k in range(16):
            e = ev[k]
            hf = jnp.full((16,), hbf + float(k), jnp.float32)
            mm, aa = [], []
            for j in range(nj):
                rv = buf[g * 16 + k, pl.ds(j * 16, 16)]
                p = rv * e
                upd = p > m[j]
                mm.append(jnp.where(upd, p, m[j]))
                aa.append(jnp.where(upd, hf, a[j]))
            m, a = tuple(mm), tuple(aa)
        return m, a

    return lax.fori_loop(0, SC_RB // 16, g_body, (m, a))


def _sc_part(emb1d, r_embedding):
    nch = SC_ROWS // SC_RB
    mesh = plsc.VectorSubcoreMesh(core_axis_name="c", subcore_axis_name="s")

    @functools.partial(
        pl.kernel,
        mesh=mesh,
        out_type=[
            jax.ShapeDtypeStruct((N,), jnp.float32),
            jax.ShapeDtypeStruct((N,), jnp.float32),
        ],
        scratch_types=[
            pltpu.VMEM((SC_ROWS,), jnp.float32),
            pltpu.VMEM((2, SC_RB, CW), jnp.float32),
            pltpu.VMEM((CW,), jnp.float32),
            pltpu.VMEM((CW,), jnp.float32),
            pltpu.SemaphoreType.DMA,
            pltpu.SemaphoreType.DMA,
        ],
    )
    def sc_kernel(emb_hbm, r_hbm, val_hbm, arg_hbm,
                  emb_v, rbuf, val_v, arg_v, sem0, sem1):
        wid = lax.axis_index("s") * 2 + lax.axis_index("c")
        c0 = wid * CW
        sems = (sem0, sem1)
        pltpu.sync_copy(emb_hbm.at[pl.ds(0, SC_ROWS)], emb_v)
        for b in range(2):
            pltpu.async_copy(
                r_hbm.at[pl.ds(b * SC_RB, SC_RB), pl.ds(c0, CW)],
                rbuf.at[b], sems[b])

        nj = CW // 16
        m0 = tuple(jnp.zeros((16,), jnp.float32) for _ in range(nj))
        a0 = tuple(jnp.zeros((16,), jnp.float32) for _ in range(nj))

        def pair_body(gp, carry):
            m, a = carry
            for b in range(2):
                g = gp * 2 + b
                pltpu.make_async_copy(
                    r_hbm.at[pl.ds(0, SC_RB), pl.ds(c0, CW)],
                    rbuf.at[b], sems[b]).wait()
                m, a = _sc_compute(rbuf.at[b], emb_v, g * SC_RB, m, a)

                @pl.when(g + 2 < nch)
                def _():
                    pltpu.async_copy(
                        r_hbm.at[pl.ds((g + 2) * SC_RB, SC_RB), pl.ds(c0, CW)],
                        rbuf.at[b], sems[b])
            return m, a

        m, a = lax.fori_loop(0, nch // 2, pair_body, (m0, a0))
        for j in range(nj):
            val_v[pl.ds(j * 16, 16)] = m[j]
            arg_v[pl.ds(j * 16, 16)] = a[j]
        pltpu.sync_copy(val_v, val_hbm.at[pl.ds(c0, CW)])
        pltpu.sync_copy(arg_v, arg_hbm.at[pl.ds(c0, CW)])

    return sc_kernel(emb1d, r_embedding)


def kernel(embedding, r_embedding):
    if SC_ROWS == 0:
        val, arg = _tc_part(embedding, r_embedding)
    elif SC_ROWS == N:
        sval, sarg = _sc_part(embedding.reshape(N), r_embedding)
        val, arg = sval[None, :], sarg[None, :]
    else:
        sval, sarg = _sc_part(embedding.reshape(N), r_embedding)
        tval, targ = _tc_part(embedding, r_embedding)
        val, arg = _merge(sval[None, :], sarg[None, :], tval, targ)
    return val, arg[0]
